# baseline (speedup 1.0000x reference)
import functools

import jax
import jax.numpy as jnp
from jax import lax
from jax.experimental import pallas as pl
from jax.experimental.pallas import tpu as pltpu
from jax.experimental.pallas import tpu_sc as plsc

NUM_ITEMS = 100000
NUM_GENRES = 18
EMB = 64
ITEM_D = 5 * EMB
ITEM_P = ITEM_D // 2
YEAR_P = EMB // 2
COMB_D = ITEM_D + EMB
OUT_D = COMB_D + EMB
BATCH = 16384

_NC = 2
_NS = 16
_NW = _NC * _NS
_BPW = BATCH // _NW
_G = _BPW // 16


def _conv_body(src_ref, dst_ref):
    x = src_ref[...]
    u = jax.lax.bitcast_convert_type(x, jnp.uint32)
    r = u + jnp.uint32(0x7FFF) + ((u >> jnp.uint32(16)) & jnp.uint32(1))
    h = x.shape[0] // 2
    hi = r[:h, :] & jnp.uint32(0xFFFF0000)
    lo = r[h:, :] >> jnp.uint32(16)
    packed = jax.lax.bitcast_convert_type(hi | lo, jnp.float32)
    dst_ref[...] = packed.T


def _convert_table(table_t, n_rows, d, row_blk):
    n_blk = (n_rows + row_blk - 1) // row_blk
    return pl.pallas_call(
        _conv_body,
        grid=(n_blk,),
        in_specs=[pl.BlockSpec((d, row_blk), lambda i: (0, i))],
        out_specs=pl.BlockSpec((row_blk, d // 2), lambda i: (i, 0)),
        out_shape=jax.ShapeDtypeStruct((n_rows, d // 2), jnp.float32),
    )(table_t)


_YOFF = 51200
_YQB = 2048


def _year_conv_body(a_ref, b_ref, out_ref):
    out_ref[:, :EMB] = a_ref[...].T
    out_ref[:, EMB:] = b_ref[...].T


def _convert_year(year_t):
    return pl.pallas_call(
        _year_conv_body,
        grid=(_YOFF // _YQB,),
        in_specs=[
            pl.BlockSpec((EMB, _YQB), lambda i: (0, i)),
            pl.BlockSpec(
                (EMB, _YQB),
                lambda i: (0, jnp.minimum(
                    _YOFF // _YQB + i,
                    (NUM_ITEMS + _YQB - 1) // _YQB - 1))),
        ],
        out_specs=pl.BlockSpec((_YQB, 2 * EMB), lambda i: (i, 0)),
        out_shape=jax.ShapeDtypeStruct((_YOFF, 2 * EMB), jnp.float32),
    )(year_t, year_t)


def _make_sc_gather(width, chunk_rows, fold=None):
    n_chunk = _BPW // chunk_rows
    n_grp = chunk_rows // 16

    def body(idx_hbm, table_hbm, out_hbm, idx_v, rows_v, sem):
        wid = lax.axis_index("s") * _NC + lax.axis_index("c")

        def chunk(c, carry):
            base = wid * _BPW + c * chunk_rows
            pltpu.sync_copy(idx_hbm.at[pl.ds(base, chunk_rows)], idx_v)

            def issue(g, carry2):
                iv = idx_v[pl.ds(g * 16, 16)]
                if fold is not None:
                    iv = jnp.where(iv >= fold, iv - fold, iv)
                for r in range(16):
                    pltpu.async_copy(
                        table_hbm.at[pl.ds(iv[r], 1)],
                        rows_v.at[pl.ds(g * 16 + r, 1)], sem)
                return carry2

            lax.fori_loop(0, n_grp, issue, 0)

            def drain(g, carry2):
                for r in range(16):
                    pltpu.make_async_copy(
                        table_hbm.at[pl.ds(0, 1)],
                        rows_v.at[pl.ds(0, 1)], sem).wait()
                return carry2

            lax.fori_loop(0, n_grp, drain, 0)
            pltpu.sync_copy(rows_v, out_hbm.at[pl.ds(base, chunk_rows)])
            return carry

        lax.fori_loop(0, n_chunk, chunk, 0)

    return pl.kernel(
        body,
        out_type=jax.ShapeDtypeStruct((BATCH, width), jnp.float32),
        mesh=plsc.VectorSubcoreMesh(core_axis_name="c", subcore_axis_name="s"),
        compiler_params=pltpu.CompilerParams(needs_layout_passes=False),
        scratch_types=[
            pltpu.VMEM((chunk_rows,), jnp.int32),
            pltpu.VMEM((chunk_rows, width), jnp.float32),
            pltpu.SemaphoreType.DMA,
        ],
    )


_sc_gather_item = _make_sc_gather(ITEM_P, 256)
_sc_gather_year = _make_sc_gather(2 * EMB, 256, fold=_YOFF)


_TB = 2048


def _unpack_t(packed):
    u = jax.lax.bitcast_convert_type(packed, jnp.uint32)
    hi = jax.lax.bitcast_convert_type(u & jnp.uint32(0xFFFF0000), jnp.float32)
    lo = jax.lax.bitcast_convert_type(u << jnp.uint32(16), jnp.float32)
    return hi.T, lo.T


def _assemble_body(item_ref, year_ref, idxf_ref, idxyf_ref, genres_t_ref,
                   w_ref, b_ref, out_ref):
    sel = (idxf_ref[0] != 0.0).astype(jnp.float32)
    it_hi, it_lo = _unpack_t(item_ref[...])
    out_ref[pl.ds(0, ITEM_P), :] = it_hi * sel
    out_ref[pl.ds(ITEM_P, ITEM_P), :] = it_lo * sel
    ysel = (idxyf_ref[0] >= float(_YOFF)).astype(jnp.float32)
    y_t = year_ref[...].T
    y_val = y_t[:EMB, :] * (1.0 - ysel) + y_t[EMB:, :] * ysel
    out_ref[pl.ds(ITEM_D, EMB), :] = y_val
    go_t = jnp.dot(w_ref[...].T, genres_t_ref[...],
                   preferred_element_type=jnp.float32)
    out_ref[pl.ds(COMB_D, EMB), :] = go_t + b_ref[...].T


def _assemble(item_emb, year_emb, idxf3, idxyf3, genres_t, w, b2d):
    return pl.pallas_call(
        _assemble_body,
        grid=(BATCH // _TB,),
        in_specs=[
            pl.BlockSpec((_TB, ITEM_P), lambda i: (i, 0)),
            pl.BlockSpec((_TB, 2 * EMB), lambda i: (i, 0)),
            pl.BlockSpec((1, 1, _TB), lambda i: (i, 0, 0)),
            pl.BlockSpec((1, 1, _TB), lambda i: (i, 0, 0)),
            pl.BlockSpec((NUM_GENRES, _TB), lambda i: (0, i)),
            pl.BlockSpec((NUM_GENRES, EMB), lambda i: (0, 0)),
            pl.BlockSpec((1, EMB), lambda i: (0, 0)),
        ],
        out_specs=pl.BlockSpec((OUT_D, _TB), lambda i: (0, i)),
        out_shape=jax.ShapeDtypeStruct((OUT_D, BATCH), jnp.float32),
    )(item_emb, year_emb, idxf3, idxyf3, genres_t, w, b2d)


def kernel(data, item_table, year_table, genre_W, genre_b):
    item_idx = data[:, 0, 0].astype(jnp.int32)
    year_idx = data[:, 0, 1].astype(jnp.int32)
    idxf3 = data[:, 0, 0].reshape(BATCH // _TB, 1, _TB)
    idxyf3 = data[:, 0, 1].reshape(BATCH // _TB, 1, _TB)
    genres_t = data[:, 0, 2:].T

    year_pk = _convert_year(year_table.T)
    year_emb = _sc_gather_year(year_idx, year_pk)
    item_bf = _convert_table(item_table.T, NUM_ITEMS + 1, ITEM_D, 8192)
    item_emb = _sc_gather_item(item_idx, item_bf)

    out_t = _assemble(item_emb, year_emb, idxf3, idxyf3, genres_t, genre_W,
                      genre_b.reshape(1, EMB))
    return out_t.T


# speedup vs baseline: 1.0386x; 1.0386x over previous
import functools

import jax
import jax.numpy as jnp
from jax import lax
from jax.experimental import pallas as pl
from jax.experimental.pallas import tpu as pltpu
from jax.experimental.pallas import tpu_sc as plsc

NUM_ITEMS = 100000
NUM_GENRES = 18
EMB = 64
ITEM_D = 5 * EMB
ITEM_P = ITEM_D // 2
YEAR_P = EMB // 2
COMB_D = ITEM_D + EMB
OUT_D = COMB_D + EMB
BATCH = 16384

_NC = 2
_NS = 16
_NW = _NC * _NS
_BPW = BATCH // _NW
_G = _BPW // 16


def _conv_body(src_ref, dst_ref):
    x = src_ref[...]
    u = jax.lax.bitcast_convert_type(x, jnp.uint32)
    r = u + jnp.uint32(0x7FFF) + ((u >> jnp.uint32(16)) & jnp.uint32(1))
    h = x.shape[0] // 2
    hi = r[:h, :] & jnp.uint32(0xFFFF0000)
    lo = r[h:, :] >> jnp.uint32(16)
    packed = jax.lax.bitcast_convert_type(hi | lo, jnp.float32)
    dst_ref[...] = packed.T


def _convert_table(table_t, n_rows, d, row_blk):
    n_blk = (n_rows + row_blk - 1) // row_blk
    return pl.pallas_call(
        _conv_body,
        grid=(n_blk,),
        in_specs=[pl.BlockSpec((d, row_blk), lambda i: (0, i))],
        out_specs=pl.BlockSpec((row_blk, d // 2), lambda i: (i, 0)),
        out_shape=jax.ShapeDtypeStruct((n_rows, d // 2), jnp.float32),
    )(table_t)


_YOFF = 57344
_YQB = 8192


def _year_conv_body(a_ref, b_ref, out_ref):
    out_ref[:, :EMB] = a_ref[...].T
    out_ref[:, EMB:] = b_ref[...].T


def _convert_year(year_t):
    return pl.pallas_call(
        _year_conv_body,
        grid=(_YOFF // _YQB,),
        in_specs=[
            pl.BlockSpec((EMB, _YQB), lambda i: (0, i)),
            pl.BlockSpec(
                (EMB, _YQB),
                lambda i: (0, jnp.minimum(
                    _YOFF // _YQB + i,
                    (NUM_ITEMS + _YQB - 1) // _YQB - 1))),
        ],
        out_specs=pl.BlockSpec((_YQB, 2 * EMB), lambda i: (i, 0)),
        out_shape=jax.ShapeDtypeStruct((_YOFF, 2 * EMB), jnp.float32),
    )(year_t, year_t)


def _make_sc_gather(width, chunk_rows, fold=None):
    n_chunk = _BPW // chunk_rows
    n_grp = chunk_rows // 16

    def body(idx_hbm, table_hbm, out_hbm, idx_v, rows_v, sem):
        wid = lax.axis_index("s") * _NC + lax.axis_index("c")

        def chunk(c, carry):
            base = wid * _BPW + c * chunk_rows
            pltpu.sync_copy(idx_hbm.at[pl.ds(base, chunk_rows)], idx_v)

            def issue(g, carry2):
                iv = idx_v[pl.ds(g * 16, 16)]
                if fold is not None:
                    iv = jnp.where(iv >= fold, iv - fold, iv)
                for r in range(16):
                    pltpu.async_copy(
                        table_hbm.at[pl.ds(iv[r], 1)],
                        rows_v.at[pl.ds(g * 16 + r, 1)], sem)
                return carry2

            lax.fori_loop(0, n_grp, issue, 0)

            def drain(g, carry2):
                for r in range(16):
                    pltpu.make_async_copy(
                        table_hbm.at[pl.ds(0, 1)],
                        rows_v.at[pl.ds(0, 1)], sem).wait()
                return carry2

            lax.fori_loop(0, n_grp, drain, 0)
            pltpu.sync_copy(rows_v, out_hbm.at[pl.ds(base, chunk_rows)])
            return carry

        lax.fori_loop(0, n_chunk, chunk, 0)

    return pl.kernel(
        body,
        out_type=jax.ShapeDtypeStruct((BATCH, width), jnp.float32),
        mesh=plsc.VectorSubcoreMesh(core_axis_name="c", subcore_axis_name="s"),
        compiler_params=pltpu.CompilerParams(needs_layout_passes=False),
        scratch_types=[
            pltpu.VMEM((chunk_rows,), jnp.int32),
            pltpu.VMEM((chunk_rows, width), jnp.float32),
            pltpu.SemaphoreType.DMA,
        ],
    )


_sc_gather_item = _make_sc_gather(ITEM_P, 256)
_sc_gather_year = _make_sc_gather(2 * EMB, 256, fold=_YOFF)


_TB = 2048


def _unpack_t(packed):
    u = jax.lax.bitcast_convert_type(packed, jnp.uint32)
    hi = jax.lax.bitcast_convert_type(u & jnp.uint32(0xFFFF0000), jnp.float32)
    lo = jax.lax.bitcast_convert_type(u << jnp.uint32(16), jnp.float32)
    return hi.T, lo.T


def _assemble_body(item_ref, year_ref, idxf_ref, idxyf_ref, genres_t_ref,
                   w_ref, b_ref, out_ref):
    sel = (idxf_ref[0] != 0.0).astype(jnp.float32)
    it_hi, it_lo = _unpack_t(item_ref[...])
    out_ref[pl.ds(0, ITEM_P), :] = it_hi * sel
    out_ref[pl.ds(ITEM_P, ITEM_P), :] = it_lo * sel
    ysel = (idxyf_ref[0] >= float(_YOFF)).astype(jnp.float32)
    y_t = year_ref[...].T
    y_val = y_t[:EMB, :] * (1.0 - ysel) + y_t[EMB:, :] * ysel
    out_ref[pl.ds(ITEM_D, EMB), :] = y_val
    go_t = jnp.dot(w_ref[...].T, genres_t_ref[...],
                   preferred_element_type=jnp.float32)
    out_ref[pl.ds(COMB_D, EMB), :] = go_t + b_ref[...].T


def _assemble(item_emb, year_emb, idxf3, idxyf3, genres_t, w, b2d):
    return pl.pallas_call(
        _assemble_body,
        grid=(BATCH // _TB,),
        in_specs=[
            pl.BlockSpec((_TB, ITEM_P), lambda i: (i, 0)),
            pl.BlockSpec((_TB, 2 * EMB), lambda i: (i, 0)),
            pl.BlockSpec((1, 1, _TB), lambda i: (i, 0, 0)),
            pl.BlockSpec((1, 1, _TB), lambda i: (i, 0, 0)),
            pl.BlockSpec((NUM_GENRES, _TB), lambda i: (0, i)),
            pl.BlockSpec((NUM_GENRES, EMB), lambda i: (0, 0)),
            pl.BlockSpec((1, EMB), lambda i: (0, 0)),
        ],
        out_specs=pl.BlockSpec((OUT_D, _TB), lambda i: (0, i)),
        out_shape=jax.ShapeDtypeStruct((OUT_D, BATCH), jnp.float32),
    )(item_emb, year_emb, idxf3, idxyf3, genres_t, w, b2d)


def kernel(data, item_table, year_table, genre_W, genre_b):
    item_idx = data[:, 0, 0].astype(jnp.int32)
    year_idx = data[:, 0, 1].astype(jnp.int32)
    idxf3 = data[:, 0, 0].reshape(BATCH // _TB, 1, _TB)
    idxyf3 = data[:, 0, 1].reshape(BATCH // _TB, 1, _TB)
    genres_t = data[:, 0, 2:].T

    year_pk = _convert_year(year_table.T)
    year_emb = _sc_gather_year(year_idx, year_pk)
    item_bf = _convert_table(item_table.T, NUM_ITEMS + 1, ITEM_D, 8192)
    item_emb = _sc_gather_item(item_idx, item_bf)

    out_t = _assemble(item_emb, year_emb, idxf3, idxyf3, genres_t, genre_W,
                      genre_b.reshape(1, EMB))
    return out_t.T


# speedup vs baseline: 1.0467x; 1.0079x over previous
import functools

import jax
import jax.numpy as jnp
from jax import lax
from jax.experimental import pallas as pl
from jax.experimental.pallas import tpu as pltpu
from jax.experimental.pallas import tpu_sc as plsc

NUM_ITEMS = 100000
NUM_GENRES = 18
EMB = 64
ITEM_D = 5 * EMB
ITEM_P = ITEM_D // 2
YEAR_P = EMB // 2
COMB_D = ITEM_D + EMB
OUT_D = COMB_D + EMB
BATCH = 16384

_NC = 2
_NS = 16
_NW = _NC * _NS
_BPW = BATCH // _NW
_G = _BPW // 16


def _conv_body(src_ref, dst_ref):
    x = src_ref[...]
    u = jax.lax.bitcast_convert_type(x, jnp.uint32)
    r = u + jnp.uint32(0x7FFF) + ((u >> jnp.uint32(16)) & jnp.uint32(1))
    h = x.shape[0] // 2
    hi = r[:h, :] & jnp.uint32(0xFFFF0000)
    lo = r[h:, :] >> jnp.uint32(16)
    packed = jax.lax.bitcast_convert_type(hi | lo, jnp.float32)
    dst_ref[...] = packed.T


def _convert_table(table_t, n_rows, d, row_blk):
    n_blk = (n_rows + row_blk - 1) // row_blk
    return pl.pallas_call(
        _conv_body,
        grid=(n_blk,),
        in_specs=[pl.BlockSpec((d, row_blk), lambda i: (0, i))],
        out_specs=pl.BlockSpec((row_blk, d // 2), lambda i: (i, 0)),
        out_shape=jax.ShapeDtypeStruct((n_rows, d // 2), jnp.float32),
    )(table_t)


_YOFF = 57344
_YQB = 8192


def _year_conv_body(a_ref, b_ref, out_ref):
    out_ref[:, :EMB] = a_ref[...].T
    out_ref[:, EMB:] = b_ref[...].T


def _convert_year(year_t):
    return pl.pallas_call(
        _year_conv_body,
        grid=(_YOFF // _YQB,),
        in_specs=[
            pl.BlockSpec((EMB, _YQB), lambda i: (0, i)),
            pl.BlockSpec(
                (EMB, _YQB),
                lambda i: (0, jnp.minimum(
                    _YOFF // _YQB + i,
                    (NUM_ITEMS + _YQB - 1) // _YQB - 1))),
        ],
        out_specs=pl.BlockSpec((_YQB, 2 * EMB), lambda i: (i, 0)),
        out_shape=jax.ShapeDtypeStruct((_YOFF, 2 * EMB), jnp.float32),
    )(year_t, year_t)


def _make_sc_gather(width, chunk_rows, fold=None):
    n_chunk = _BPW // chunk_rows
    n_grp = chunk_rows // 16

    def body(idx_hbm, table_hbm, out_hbm, idx_v, rows_v, sem):
        wid = lax.axis_index("s") * _NC + lax.axis_index("c")

        def chunk(c, carry):
            base = wid * _BPW + c * chunk_rows
            pltpu.sync_copy(idx_hbm.at[pl.ds(base, chunk_rows)], idx_v)

            def issue(g, carry2):
                iv = idx_v[pl.ds(g * 16, 16)]
                if fold is not None:
                    iv = jnp.where(iv >= fold, iv - fold, iv)
                for r in range(16):
                    pltpu.async_copy(
                        table_hbm.at[pl.ds(iv[r], 1)],
                        rows_v.at[pl.ds(g * 16 + r, 1)], sem)
                return carry2

            lax.fori_loop(0, n_grp, issue, 0)

            def drain(g, carry2):
                for r in range(16):
                    pltpu.make_async_copy(
                        table_hbm.at[pl.ds(0, 1)],
                        rows_v.at[pl.ds(0, 1)], sem).wait()
                return carry2

            lax.fori_loop(0, n_grp, drain, 0)
            pltpu.sync_copy(rows_v, out_hbm.at[pl.ds(base, chunk_rows)])
            return carry

        lax.fori_loop(0, n_chunk, chunk, 0)

    return pl.kernel(
        body,
        out_type=jax.ShapeDtypeStruct((BATCH, width), jnp.float32),
        mesh=plsc.VectorSubcoreMesh(core_axis_name="c", subcore_axis_name="s"),
        compiler_params=pltpu.CompilerParams(needs_layout_passes=False),
        scratch_types=[
            pltpu.VMEM((chunk_rows,), jnp.int32),
            pltpu.VMEM((chunk_rows, width), jnp.float32),
            pltpu.SemaphoreType.DMA,
        ],
    )


_sc_gather_item = _make_sc_gather(ITEM_P, 256)
_sc_gather_year = _make_sc_gather(2 * EMB, 256, fold=_YOFF)


_TB = 4096


def _unpack_t(packed):
    u = jax.lax.bitcast_convert_type(packed, jnp.uint32)
    hi = jax.lax.bitcast_convert_type(u & jnp.uint32(0xFFFF0000), jnp.float32)
    lo = jax.lax.bitcast_convert_type(u << jnp.uint32(16), jnp.float32)
    return hi.T, lo.T


def _assemble_body(item_ref, year_ref, idxf_ref, idxyf_ref, genres_t_ref,
                   w_ref, b_ref, out_ref):
    sel = (idxf_ref[0] != 0.0).astype(jnp.float32)
    it_hi, it_lo = _unpack_t(item_ref[...])
    out_ref[pl.ds(0, ITEM_P), :] = it_hi * sel
    out_ref[pl.ds(ITEM_P, ITEM_P), :] = it_lo * sel
    ysel = (idxyf_ref[0] >= float(_YOFF)).astype(jnp.float32)
    y_t = year_ref[...].T
    y_val = y_t[:EMB, :] * (1.0 - ysel) + y_t[EMB:, :] * ysel
    out_ref[pl.ds(ITEM_D, EMB), :] = y_val
    go_t = jnp.dot(w_ref[...].T, genres_t_ref[...],
                   preferred_element_type=jnp.float32)
    out_ref[pl.ds(COMB_D, EMB), :] = go_t + b_ref[...].T


def _assemble(item_emb, year_emb, idxf3, idxyf3, genres_t, w, b2d):
    return pl.pallas_call(
        _assemble_body,
        grid=(BATCH // _TB,),
        in_specs=[
            pl.BlockSpec((_TB, ITEM_P), lambda i: (i, 0)),
            pl.BlockSpec((_TB, 2 * EMB), lambda i: (i, 0)),
            pl.BlockSpec((1, 1, _TB), lambda i: (i, 0, 0)),
            pl.BlockSpec((1, 1, _TB), lambda i: (i, 0, 0)),
            pl.BlockSpec((NUM_GENRES, _TB), lambda i: (0, i)),
            pl.BlockSpec((NUM_GENRES, EMB), lambda i: (0, 0)),
            pl.BlockSpec((1, EMB), lambda i: (0, 0)),
        ],
        out_specs=pl.BlockSpec((OUT_D, _TB), lambda i: (0, i)),
        out_shape=jax.ShapeDtypeStruct((OUT_D, BATCH), jnp.float32),
    )(item_emb, year_emb, idxf3, idxyf3, genres_t, w, b2d)


def kernel(data, item_table, year_table, genre_W, genre_b):
    item_idx = data[:, 0, 0].astype(jnp.int32)
    year_idx = data[:, 0, 1].astype(jnp.int32)
    idxf3 = data[:, 0, 0].reshape(BATCH // _TB, 1, _TB)
    idxyf3 = data[:, 0, 1].reshape(BATCH // _TB, 1, _TB)
    genres_t = data[:, 0, 2:].T

    year_pk = _convert_year(year_table.T)
    year_emb = _sc_gather_year(year_idx, year_pk)
    item_bf = _convert_table(item_table.T, NUM_ITEMS + 1, ITEM_D, 10240)
    item_emb = _sc_gather_item(item_idx, item_bf)

    out_t = _assemble(item_emb, year_emb, idxf3, idxyf3, genres_t, genre_W,
                      genre_b.reshape(1, EMB))
    return out_t.T
